# Initial kernel scaffold; baseline (speedup 1.0000x reference)
import jax
import jax.numpy as jnp
from jax.experimental import pallas as pl


def kernel(x, edge_index, edge_attr, batch, params):
    raise NotImplementedError("write your pallas kernel here")


# speedup vs baseline: 1.5096x; 1.5096x over previous
import functools

import jax
import jax.numpy as jnp
from jax.experimental import pallas as pl


def _relu(v):
    return jnp.maximum(v, 0.0)


def _layer(p, h, src, dst, edge_attr, n):
    cin = h.shape[1]
    w1 = p["mlp1"]["w"]
    wa = w1[:cin]
    wb = w1[cin : 2 * cin]
    wc = w1[2 * cin :]
    anode = h @ (wa - wb)
    bnode = h @ wb
    hrelu = _relu(edge_attr @ p["em1"]["w"] + p["em1"]["b"])
    eac = hrelu @ (p["em2"]["w"] @ wc) + (p["em2"]["b"] @ wc + p["mlp1"]["b"])
    msg = _relu(anode[dst] + bnode[src] + eac)
    w1w7 = jnp.tanh(h @ p["mlp5"]["w"] + p["mlp5"]["b"]) * p["mlp7"]["w"][:, 0]
    w2 = jnp.tanh(msg @ p["mlp6"]["w"] + p["mlp6"]["b"])
    logit = jnp.sum(w1w7[dst] * w2, axis=-1) + p["mlp7"]["b"][0]
    pexp = jnp.exp(logit)
    s = jax.ops.segment_sum(pexp, src, num_segments=n)
    w = pexp / s[src]
    agg = jax.ops.segment_sum(msg * w[:, None], dst, num_segments=n)
    xo = _relu(h @ p["mlp2"]["w"] + p["mlp2"]["b"])
    cat = jnp.concatenate([xo, agg], axis=1)
    g1 = jax.nn.sigmoid(cat @ p["mlp3"]["w"] + p["mlp3"]["b"])
    g2 = jax.nn.sigmoid(cat @ p["mlp4"]["w"] + p["mlp4"]["b"])
    return g1 * agg + g2 * xo


def _gatt(p, h, batch, g):
    logit = (h @ p["w"] + p["b"])[:, 0]
    m = jax.ops.segment_max(logit, batch, num_segments=g)
    e = jnp.exp(logit - m[batch])
    s = jax.ops.segment_sum(e, batch, num_segments=g)
    gate = e / (s[batch] + 1e-16)
    return jax.ops.segment_sum(gate[:, None] * h, batch, num_segments=g)


def _head_kernel(z_ref, w1, b1, w2, b2, w3, b3, w4, b4, o_ref):
    z = z_ref[...]
    z = jnp.maximum(z @ w1[...] + b1[...], 0.0)
    z = jnp.maximum(z @ w2[...] + b2[...], 0.0)
    z = jnp.maximum(z @ w3[...] + b3[...], 0.0)
    z = z @ w4[...] + b4[...]
    o_ref[...] = z


def _head(params, z):
    g = z.shape[0]
    args = []
    for k in ("lin1", "lin2", "lin3", "lin4"):
        args += [params[k]["w"], params[k]["b"]]
    out = pl.pallas_call(
        _head_kernel,
        out_shape=jax.ShapeDtypeStruct((g, 1), jnp.float32),
    )(z, *args)
    return out[:, 0]


def kernel(x, edge_index, edge_attr, batch, params):
    n = x.shape[0]
    g = 64
    src = edge_index[0]
    dst = edge_index[1]
    h = _relu(_layer(params["conv1"], x, src, dst, edge_attr, n))
    h = _relu(_layer(params["conv2"], h, src, dst, edge_attr, n))
    x1 = _gatt(params["gate1"], h, batch, g)
    h = _relu(_layer(params["conv3"], h, src, dst, edge_attr, n))
    h = _relu(_layer(params["conv4"], h, src, dst, edge_attr, n))
    x2 = _gatt(params["gate2"], h, batch, g)
    z = jnp.concatenate([x1, x2], axis=1)
    return _head(params, z)


# speedup vs baseline: 4.6603x; 3.0871x over previous
import functools

import jax
import jax.numpy as jnp
from jax import lax
from jax.experimental import pallas as pl
from jax.experimental.pallas import tpu as pltpu
from jax.experimental.pallas import tpu_sc as plsc

_N = 50000
_NP = 50048
_E = 1600000
_G = 64
_NW = 32
_EPW = _E // _NW
_CHUNK = 80
_NCH = _EPW // _CHUNK
_RPT = _NP // 16
_NB = 2000
_EB = 4000

f32 = jnp.float32
i32 = jnp.int32


def _mesh():
    return plsc.VectorSubcoreMesh(core_axis_name="c", subcore_axis_name="s")


_SC_PARAMS = pltpu.CompilerParams(use_tc_tiling_on_sc=False)


def _sc_gather2(dst, src, dtab, stab):
    dw = dtab.shape[1]
    sw = stab.shape[1]

    def body(dst_h, src_h, dt_h, st_h, ge_h, se_h, dix, six, dbuf, sbuf, s1, s2):
        wid = lax.axis_index("s") * 2 + lax.axis_index("c")
        base = wid * _EPW

        def it(i, _):
            off = base + i * _CHUNK
            pltpu.sync_copy(dst_h.at[pl.ds(off, _CHUNK)], dix)
            pltpu.sync_copy(src_h.at[pl.ds(off, _CHUNK)], six)
            c1 = pltpu.async_copy(dt_h.at[dix], dbuf, s1)
            c2 = pltpu.async_copy(st_h.at[six], sbuf, s2)
            c1.wait()
            c2.wait()
            pltpu.sync_copy(dbuf, ge_h.at[pl.ds(off, _CHUNK), :])
            pltpu.sync_copy(sbuf, se_h.at[pl.ds(off, _CHUNK), :])
            return 0

        lax.fori_loop(0, _NCH, it, 0)

    return pl.kernel(
        body,
        out_type=[
            jax.ShapeDtypeStruct((_E, dw), f32),
            jax.ShapeDtypeStruct((_E, sw), f32),
        ],
        mesh=_mesh(),
        compiler_params=_SC_PARAMS,
        scratch_types=[
            pltpu.VMEM((_CHUNK,), i32),
            pltpu.VMEM((_CHUNK,), i32),
            pltpu.VMEM((_CHUNK, dw), f32),
            pltpu.VMEM((_CHUNK, sw), f32),
            pltpu.SemaphoreType.DMA,
            pltpu.SemaphoreType.DMA,
        ],
    )(dst, src, dtab, stab)


def _sc_gather1(idx, tab):
    w = tab.shape[1]

    def body(idx_h, t_h, o_h, ib, buf, s1):
        wid = lax.axis_index("s") * 2 + lax.axis_index("c")
        base = wid * _EPW

        def it(i, _):
            off = base + i * _CHUNK
            pltpu.sync_copy(idx_h.at[pl.ds(off, _CHUNK)], ib)
            pltpu.async_copy(t_h.at[ib], buf, s1).wait()
            pltpu.sync_copy(buf, o_h.at[pl.ds(off, _CHUNK), :])
            return 0

        lax.fori_loop(0, _NCH, it, 0)

    return pl.kernel(
        body,
        out_type=jax.ShapeDtypeStruct((_E, w), f32),
        mesh=_mesh(),
        compiler_params=_SC_PARAMS,
        scratch_types=[
            pltpu.VMEM((_CHUNK,), i32),
            pltpu.VMEM((_CHUNK, w), f32),
            pltpu.SemaphoreType.DMA,
        ],
    )(idx, tab)


def _sc_scatter_add(idx, vals, zeros):
    w = vals.shape[1]

    def body(idx_h, val_h, z_h, out_h, ib, vb, acc_sh):
        c = lax.axis_index("c")
        s = lax.axis_index("s")
        wid = s * 2 + c
        r0 = s * _RPT
        pltpu.sync_copy(z_h.at[pl.ds(r0, _RPT), :], acc_sh.at[pl.ds(r0, _RPT), :])
        plsc.subcore_barrier()
        base = wid * _EPW

        def it(i, _):
            off = base + i * _CHUNK
            pltpu.sync_copy(idx_h.at[pl.ds(off, _CHUNK)], ib)
            pltpu.sync_copy(val_h.at[pl.ds(off, _CHUNK), :], vb)
            pltpu.sync_copy(vb, acc_sh.at[ib], add=True)
            return 0

        lax.fori_loop(0, _NCH, it, 0)
        plsc.subcore_barrier()
        pltpu.sync_copy(acc_sh.at[pl.ds(r0, _RPT), :], out_h.at[c, pl.ds(r0, _RPT), :])

    return pl.kernel(
        body,
        out_type=jax.ShapeDtypeStruct((2, _NP, w), f32),
        mesh=_mesh(),
        compiler_params=_SC_PARAMS,
        scratch_types=[
            pltpu.VMEM((_CHUNK,), i32),
            pltpu.VMEM((_CHUNK, w), f32),
            pltpu.VMEM_SHARED((_NP, w), f32),
        ],
    )(idx, vals, zeros)


def _full(shape):
    return pl.BlockSpec(shape, lambda i: tuple(0 for _ in shape))


def _prep(h, wd, wb, w5, b5, w7):
    cin = h.shape[1]
    cout = wd.shape[1]

    def body(h_ref, wd_ref, wb_ref, w5_ref, b5_ref, w7_ref, dt_ref, st_ref):
        hb = h_ref[...]
        an = hb @ wd_ref[...]
        bn = hb @ wb_ref[...]
        w1w7 = jnp.tanh(hb @ w5_ref[...] + b5_ref[...]) * w7_ref[...]
        if cout < 16:
            padd = jnp.zeros((hb.shape[0], 16 - cout), f32)
            dt_ref[...] = jnp.concatenate([an, padd, w1w7], axis=1)
            st_ref[...] = jnp.concatenate([bn, padd], axis=1)
        else:
            dt_ref[...] = jnp.concatenate([an, w1w7], axis=1)
            st_ref[...] = bn

    return pl.pallas_call(
        body,
        grid=(_N // _NB,),
        in_specs=[
            pl.BlockSpec((_NB, cin), lambda i: (i, 0)),
            _full((cin, cout)),
            _full((cin, cout)),
            _full((cin, 16)),
            _full((1, 16)),
            _full((1, 16)),
        ],
        out_specs=[
            pl.BlockSpec((_NB, 32), lambda i: (i, 0)),
            pl.BlockSpec((_NB, 16), lambda i: (i, 0)),
        ],
        out_shape=[
            jax.ShapeDtypeStruct((_N, 32), f32),
            jax.ShapeDtypeStruct((_N, 16), f32),
        ],
    )(h, wd, wb, w5, b5, w7)


def _eac(edge_attr, e1w, e1b, kmat, kbias):
    cout = kmat.shape[1]

    def body(ea_ref, w_ref, b_ref, k_ref, kb_ref, o_ref):
        hrelu = jnp.maximum(ea_ref[...] @ w_ref[...] + b_ref[...], 0.0)
        o_ref[...] = hrelu @ k_ref[...] + kb_ref[...]

    return pl.pallas_call(
        body,
        grid=(_E // _EB,),
        in_specs=[
            pl.BlockSpec((_EB, 3), lambda i: (i, 0)),
            _full((3, 16)),
            _full((1, 16)),
            _full((16, cout)),
            _full((1, cout)),
        ],
        out_specs=pl.BlockSpec((_EB, cout), lambda i: (i, 0)),
        out_shape=jax.ShapeDtypeStruct((_E, cout), f32),
    )(edge_attr, e1w, e1b, kmat, kbias)


def _edge_msgp(ge, se, eac, w6, b6, b7):
    cout = eac.shape[1]

    def body(ge_ref, se_ref, eac_ref, w6_ref, b6_ref, b7_ref, mp_ref, pe_ref):
        a = ge_ref[...]
        msg = jnp.maximum(a[:, :cout] + se_ref[...][:, :cout] + eac_ref[...], 0.0)
        w2 = jnp.tanh(msg @ w6_ref[...] + b6_ref[...])
        logit = jnp.sum(a[:, 16:32] * w2, axis=1, keepdims=True) + b7_ref[...]
        pexp = jnp.exp(logit)
        mp_ref[...] = msg * pexp
        pe_ref[...] = jnp.broadcast_to(pexp, (pexp.shape[0], 8))

    return pl.pallas_call(
        body,
        grid=(_E // _EB,),
        in_specs=[
            pl.BlockSpec((_EB, 32), lambda i: (i, 0)),
            pl.BlockSpec((_EB, 16), lambda i: (i, 0)),
            pl.BlockSpec((_EB, cout), lambda i: (i, 0)),
            _full((cout, 16)),
            _full((1, 16)),
            _full((1, 1)),
        ],
        out_specs=[
            pl.BlockSpec((_EB, cout), lambda i: (i, 0)),
            pl.BlockSpec((_EB, 8), lambda i: (i, 0)),
        ],
        out_shape=[
            jax.ShapeDtypeStruct((_E, cout), f32),
            jax.ShapeDtypeStruct((_E, 8), f32),
        ],
    )(ge, se, eac, w6, b6, b7)


def _sinv(sp):
    def body(s_ref, o_ref):
        s = s_ref[...]
        o_ref[...] = 1.0 / (s[0] + s[1])

    return pl.pallas_call(
        body,
        grid=(_NP // _RPT,),
        in_specs=[pl.BlockSpec((2, _RPT, 8), lambda i: (0, i, 0))],
        out_specs=pl.BlockSpec((_RPT, 8), lambda i: (i, 0)),
        out_shape=jax.ShapeDtypeStruct((_NP, 8), f32),
    )(sp)


def _scale(mp, sg):
    cout = mp.shape[1]
    wpad = max(cout, 8)

    def body(mp_ref, sg_ref, o_ref):
        r = mp_ref[...] * sg_ref[...][:, 0:1]
        if cout < wpad:
            r = jnp.concatenate([r, jnp.zeros((r.shape[0], wpad - cout), f32)], axis=1)
        o_ref[...] = r

    return pl.pallas_call(
        body,
        grid=(_E // _EB,),
        in_specs=[
            pl.BlockSpec((_EB, cout), lambda i: (i, 0)),
            pl.BlockSpec((_EB, 8), lambda i: (i, 0)),
        ],
        out_specs=pl.BlockSpec((_EB, wpad), lambda i: (i, 0)),
        out_shape=jax.ShapeDtypeStruct((_E, wpad), f32),
    )(mp, sg)


def _combine(h, aggp, w2, b2, w3, b3, w4, b4):
    cin = h.shape[1]
    cout = w2.shape[1]

    wpad = max(cout, 8)

    def body(h_ref, a_ref, w2r, b2r, w3r, b3r, w4r, b4r, o_ref):
        a = a_ref[...]
        agg = (a[0] + a[1])[:, :cout]
        xo = jnp.maximum(h_ref[...] @ w2r[...] + b2r[...], 0.0)
        cat = jnp.concatenate([xo, agg], axis=1)
        g1 = jax.nn.sigmoid(cat @ w3r[...] + b3r[...])
        g2 = jax.nn.sigmoid(cat @ w4r[...] + b4r[...])
        o_ref[...] = jnp.maximum(g1 * agg + g2 * xo, 0.0)

    return pl.pallas_call(
        body,
        grid=(_N // _NB,),
        in_specs=[
            pl.BlockSpec((_NB, cin), lambda i: (i, 0)),
            pl.BlockSpec((2, _NB, wpad), lambda i: (0, i, 0)),
            _full((cin, cout)),
            _full((1, cout)),
            _full((2 * cout, 1)),
            _full((1, 1)),
            _full((2 * cout, 1)),
            _full((1, 1)),
        ],
        out_specs=pl.BlockSpec((_NB, cout), lambda i: (i, 0)),
        out_shape=jax.ShapeDtypeStruct((_N, cout), f32),
    )(h, aggp, w2, b2, w3, b3, w4, b4)


def _pool_max(h, batch2, wg, bg):
    c = h.shape[1]

    def body(h_ref, b_ref, wg_ref, bg_ref, m_ref):
        i = pl.program_id(0)
        logit = h_ref[...] @ wg_ref[...] + bg_ref[...]
        oh = b_ref[...] == lax.broadcasted_iota(i32, (_NB, _G), 1)
        masked = jnp.where(oh, logit, -1e38)
        bm = jnp.max(masked, axis=0, keepdims=True)

        @pl.when(i == 0)
        def _():
            m_ref[...] = bm

        @pl.when(i > 0)
        def _():
            m_ref[...] = jnp.maximum(m_ref[...], bm)

    return pl.pallas_call(
        body,
        grid=(_N // _NB,),
        in_specs=[
            pl.BlockSpec((_NB, c), lambda i: (i, 0)),
            pl.BlockSpec((_NB, 1), lambda i: (i, 0)),
            _full((c, 1)),
            _full((1, 1)),
        ],
        out_specs=pl.BlockSpec((1, _G), lambda i: (0, 0)),
        out_shape=jax.ShapeDtypeStruct((1, _G), f32),
    )(h, batch2, wg, bg)


def _pool_sum(h, batch2, wg, bg, m):
    c = h.shape[1]

    def body(h_ref, b_ref, wg_ref, bg_ref, m_ref, num_ref, s_ref):
        i = pl.program_id(0)
        logit = h_ref[...] @ wg_ref[...] + bg_ref[...]
        oh = b_ref[...] == lax.broadcasted_iota(i32, (_NB, _G), 1)
        ohf = oh.astype(f32)
        mnode = lax.dot_general(ohf, m_ref[...], (((1,), (1,)), ((), ())))
        e = jnp.exp(logit - mnode)
        sblk = jnp.sum(ohf * e, axis=0, keepdims=True)
        numblk = lax.dot_general(ohf, e * h_ref[...], (((0,), (0,)), ((), ())))

        @pl.when(i == 0)
        def _():
            num_ref[...] = numblk
            s_ref[...] = sblk

        @pl.when(i > 0)
        def _():
            num_ref[...] = num_ref[...] + numblk
            s_ref[...] = s_ref[...] + sblk

    return pl.pallas_call(
        body,
        grid=(_N // _NB,),
        in_specs=[
            pl.BlockSpec((_NB, c), lambda i: (i, 0)),
            pl.BlockSpec((_NB, 1), lambda i: (i, 0)),
            _full((c, 1)),
            _full((1, 1)),
            _full((1, _G)),
        ],
        out_specs=[
            pl.BlockSpec((_G, c), lambda i: (0, 0)),
            pl.BlockSpec((1, _G), lambda i: (0, 0)),
        ],
        out_shape=[
            jax.ShapeDtypeStruct((_G, c), f32),
            jax.ShapeDtypeStruct((1, _G), f32),
        ],
    )(h, batch2, wg, bg, m)


def _head(num1, s1, num2, s2, params):
    def body(n1, s1r, n2, s2r, w1, b1, w2, b2, w3, b3, w4, b4, o_ref):
        x1 = n1[...] / (jnp.transpose(s1r[...]) + 1e-16)
        x2 = n2[...] / (jnp.transpose(s2r[...]) + 1e-16)
        z = jnp.concatenate([x1, x2], axis=1)
        z = jnp.maximum(z @ w1[...] + b1[...], 0.0)
        z = jnp.maximum(z @ w2[...] + b2[...], 0.0)
        z = jnp.maximum(z @ w3[...] + b3[...], 0.0)
        o_ref[...] = z @ w4[...] + b4[...]

    args = [num1, s1, num2, s2]
    for k in ("lin1", "lin2", "lin3", "lin4"):
        args += [params[k]["w"], params[k]["b"][None, :]]
    return pl.pallas_call(
        body,
        out_shape=jax.ShapeDtypeStruct((_G, 1), f32),
    )(*args)


def _layer(p, h, src, dst, edge_attr, zeros8):
    cin = h.shape[1]
    cout = p["mlp2"]["w"].shape[1]
    w1 = p["mlp1"]["w"]
    wa = w1[:cin]
    wb = w1[cin : 2 * cin]
    wc = w1[2 * cin :]
    kmat = p["em2"]["w"] @ wc
    kbias = (p["em2"]["b"] @ wc + p["mlp1"]["b"])[None, :]
    dt, st = _prep(
        h, wa - wb, wb, p["mlp5"]["w"], p["mlp5"]["b"][None, :], p["mlp7"]["w"][:, 0][None, :]
    )
    eac = _eac(edge_attr, p["em1"]["w"], p["em1"]["b"][None, :], kmat, kbias)
    ge, se = _sc_gather2(dst, src, dt, st)
    mp, pe = _edge_msgp(ge, se, eac, p["mlp6"]["w"], p["mlp6"]["b"][None, :], p["mlp7"]["b"].reshape(1, 1))
    sp = _sc_scatter_add(src, pe, zeros8)
    sinv = _sinv(sp)
    sg = _sc_gather1(src, sinv)
    mps = _scale(mp, sg)
    aggp = _sc_scatter_add(dst, mps, jnp.zeros((_NP, max(cout, 8)), f32))
    return _combine(
        h,
        aggp,
        p["mlp2"]["w"],
        p["mlp2"]["b"][None, :],
        p["mlp3"]["w"],
        p["mlp3"]["b"][None, :],
        p["mlp4"]["w"],
        p["mlp4"]["b"][None, :],
    )


def kernel(x, edge_index, edge_attr, batch, params):
    src = edge_index[0]
    dst = edge_index[1]
    batch2 = batch[:, None]
    zeros8 = jnp.zeros((_NP, 8), f32)

    h = _layer(params["conv1"], x, src, dst, edge_attr, zeros8)
    h = _layer(params["conv2"], h, src, dst, edge_attr, zeros8)
    g1w = params["gate1"]["w"]
    g1b = params["gate1"]["b"][None, :]
    m1 = _pool_max(h, batch2, g1w, g1b)
    n1, s1 = _pool_sum(h, batch2, g1w, g1b, m1)
    h = _layer(params["conv3"], h, src, dst, edge_attr, zeros8)
    h = _layer(params["conv4"], h, src, dst, edge_attr, zeros8)
    g2w = params["gate2"]["w"]
    g2b = params["gate2"]["b"][None, :]
    m2 = _pool_max(h, batch2, g2w, g2b)
    n2, s2 = _pool_sum(h, batch2, g2w, g2b, m2)
    out = _head(n1, s1, n2, s2, params)
    return out[:, 0]


# speedup vs baseline: 6.1416x; 1.3178x over previous
import functools

import jax
import jax.numpy as jnp
from jax import lax
from jax.experimental import pallas as pl
from jax.experimental.pallas import tpu as pltpu
from jax.experimental.pallas import tpu_sc as plsc

_N = 50000
_NP = 50048
_E = 1600000
_G = 64
_NW = 32
_EPW = _E // _NW
_CHUNK = 80
_NCH = _EPW // _CHUNK
_K = 5
_KE = _K * _CHUNK
_NOUT = _EPW // _KE
_RPT = _NP // 16
_NB = 2000
_EB = 4000

f32 = jnp.float32
i32 = jnp.int32


def _mesh():
    return plsc.VectorSubcoreMesh(core_axis_name="c", subcore_axis_name="s")


_SC_PARAMS = pltpu.CompilerParams(use_tc_tiling_on_sc=False)


def _sc_gather2(dst2, src2, dtab, stab):
    dw = dtab.shape[1]
    sw = stab.shape[1]

    def body(dst_h, src_h, dt_h, st_h, ge_h, se_h, dix, six, dbuf, sbuf, s1, s2):
        wid = lax.axis_index("s") * 2 + lax.axis_index("c")
        base = wid * _EPW
        crow = wid * _NCH

        def it(i, _):
            off = base + i * _KE
            r0 = crow + i * _K
            pltpu.sync_copy(dst_h.at[pl.ds(r0, _K), :], dix)
            pltpu.sync_copy(src_h.at[pl.ds(r0, _K), :], six)
            hs = []
            for j in range(_K):
                hs.append(
                    pltpu.async_copy(
                        dt_h.at[dix.at[j]], dbuf.at[pl.ds(j * _CHUNK, _CHUNK), :], s1
                    )
                )
                hs.append(
                    pltpu.async_copy(
                        st_h.at[six.at[j]], sbuf.at[pl.ds(j * _CHUNK, _CHUNK), :], s2
                    )
                )
            for h in hs:
                h.wait()
            pltpu.sync_copy(dbuf, ge_h.at[pl.ds(off, _KE), :])
            pltpu.sync_copy(sbuf, se_h.at[pl.ds(off, _KE), :])
            return 0

        lax.fori_loop(0, _NOUT, it, 0)

    return pl.kernel(
        body,
        out_type=[
            jax.ShapeDtypeStruct((_E, dw), f32),
            jax.ShapeDtypeStruct((_E, sw), f32),
        ],
        mesh=_mesh(),
        compiler_params=_SC_PARAMS,
        scratch_types=[
            pltpu.VMEM((_K, _CHUNK), i32),
            pltpu.VMEM((_K, _CHUNK), i32),
            pltpu.VMEM((_KE, dw), f32),
            pltpu.VMEM((_KE, sw), f32),
            pltpu.SemaphoreType.DMA,
            pltpu.SemaphoreType.DMA,
        ],
    )(dst2, src2, dtab, stab)


def _sc_gather1(idx2, tab):
    w = tab.shape[1]

    def body(idx_h, t_h, o_h, ib, buf, s1):
        wid = lax.axis_index("s") * 2 + lax.axis_index("c")
        base = wid * _EPW
        crow = wid * _NCH

        def it(i, _):
            off = base + i * _KE
            r0 = crow + i * _K
            pltpu.sync_copy(idx_h.at[pl.ds(r0, _K), :], ib)
            hs = []
            for j in range(_K):
                hs.append(
                    pltpu.async_copy(
                        t_h.at[ib.at[j]], buf.at[pl.ds(j * _CHUNK, _CHUNK), :], s1
                    )
                )
            for h in hs:
                h.wait()
            pltpu.sync_copy(buf, o_h.at[pl.ds(off, _KE), :])
            return 0

        lax.fori_loop(0, _NOUT, it, 0)

    return pl.kernel(
        body,
        out_type=jax.ShapeDtypeStruct((_E, w), f32),
        mesh=_mesh(),
        compiler_params=_SC_PARAMS,
        scratch_types=[
            pltpu.VMEM((_K, _CHUNK), i32),
            pltpu.VMEM((_KE, w), f32),
            pltpu.SemaphoreType.DMA,
        ],
    )(idx2, tab)


def _sc_scatter_add(idx2, vals, zeros):
    w = vals.shape[1]

    def body(idx_h, val_h, z_h, out_h, ib, vb, acc_sh, s1):
        c = lax.axis_index("c")
        s = lax.axis_index("s")
        wid = s * 2 + c
        r0 = s * _RPT
        pltpu.sync_copy(z_h.at[pl.ds(r0, _RPT), :], acc_sh.at[pl.ds(r0, _RPT), :])
        plsc.subcore_barrier()
        base = wid * _EPW
        crow = wid * _NCH

        def it(i, _):
            off = base + i * _KE
            rr = crow + i * _K
            pltpu.sync_copy(idx_h.at[pl.ds(rr, _K), :], ib)
            pltpu.sync_copy(val_h.at[pl.ds(off, _KE), :], vb)
            hs = []
            for j in range(_K):
                hs.append(
                    pltpu.async_copy(
                        vb.at[pl.ds(j * _CHUNK, _CHUNK), :],
                        acc_sh.at[ib.at[j]],
                        s1,
                        add=True,
                    )
                )
            for h in hs:
                h.wait()
            return 0

        lax.fori_loop(0, _NOUT, it, 0)
        plsc.subcore_barrier()
        pltpu.sync_copy(acc_sh.at[pl.ds(r0, _RPT), :], out_h.at[c, pl.ds(r0, _RPT), :])

    return pl.kernel(
        body,
        out_type=jax.ShapeDtypeStruct((2, _NP, w), f32),
        mesh=_mesh(),
        compiler_params=_SC_PARAMS,
        scratch_types=[
            pltpu.VMEM((_K, _CHUNK), i32),
            pltpu.VMEM((_KE, w), f32),
            pltpu.VMEM_SHARED((_NP, w), f32),
            pltpu.SemaphoreType.DMA,
        ],
    )(idx2, vals, zeros)


def _full(shape):
    return pl.BlockSpec(shape, lambda i: tuple(0 for _ in shape))


def _prep(h, wd, wb, w5, b5, w7):
    cin = h.shape[1]
    cout = wd.shape[1]

    def body(h_ref, wd_ref, wb_ref, w5_ref, b5_ref, w7_ref, dt_ref, st_ref):
        hb = h_ref[...]
        an = hb @ wd_ref[...]
        bn = hb @ wb_ref[...]
        w1w7 = jnp.tanh(hb @ w5_ref[...] + b5_ref[...]) * w7_ref[...]
        if cout < 16:
            padd = jnp.zeros((hb.shape[0], 16 - cout), f32)
            dt_ref[...] = jnp.concatenate([an, padd, w1w7], axis=1)
            st_ref[...] = jnp.concatenate([bn, padd], axis=1)
        else:
            dt_ref[...] = jnp.concatenate([an, w1w7], axis=1)
            st_ref[...] = bn

    return pl.pallas_call(
        body,
        grid=(_N // _NB,),
        in_specs=[
            pl.BlockSpec((_NB, cin), lambda i: (i, 0)),
            _full((cin, cout)),
            _full((cin, cout)),
            _full((cin, 16)),
            _full((1, 16)),
            _full((1, 16)),
        ],
        out_specs=[
            pl.BlockSpec((_NB, 32), lambda i: (i, 0)),
            pl.BlockSpec((_NB, 16), lambda i: (i, 0)),
        ],
        out_shape=[
            jax.ShapeDtypeStruct((_N, 32), f32),
            jax.ShapeDtypeStruct((_N, 16), f32),
        ],
    )(h, wd, wb, w5, b5, w7)


def _eac(edge_attr, e1w, e1b, kmat, kbias):
    cout = kmat.shape[1]

    def body(ea_ref, w_ref, b_ref, k_ref, kb_ref, o_ref):
        hrelu = jnp.maximum(ea_ref[...] @ w_ref[...] + b_ref[...], 0.0)
        o_ref[...] = hrelu @ k_ref[...] + kb_ref[...]

    return pl.pallas_call(
        body,
        grid=(_E // _EB,),
        in_specs=[
            pl.BlockSpec((_EB, 3), lambda i: (i, 0)),
            _full((3, 16)),
            _full((1, 16)),
            _full((16, cout)),
            _full((1, cout)),
        ],
        out_specs=pl.BlockSpec((_EB, cout), lambda i: (i, 0)),
        out_shape=jax.ShapeDtypeStruct((_E, cout), f32),
    )(edge_attr, e1w, e1b, kmat, kbias)


def _edge_msgp(ge, se, eac, w6, b6, b7):
    cout = eac.shape[1]

    def body(ge_ref, se_ref, eac_ref, w6_ref, b6_ref, b7_ref, mp_ref, pe_ref):
        a = ge_ref[...]
        msg = jnp.maximum(a[:, :cout] + se_ref[...][:, :cout] + eac_ref[...], 0.0)
        w2 = jnp.tanh(msg @ w6_ref[...] + b6_ref[...])
        logit = jnp.sum(a[:, 16:32] * w2, axis=1, keepdims=True) + b7_ref[...]
        pexp = jnp.exp(logit)
        mp_ref[...] = msg * pexp
        pe_ref[...] = jnp.broadcast_to(pexp, (pexp.shape[0], 8))

    return pl.pallas_call(
        body,
        grid=(_E // _EB,),
        in_specs=[
            pl.BlockSpec((_EB, 32), lambda i: (i, 0)),
            pl.BlockSpec((_EB, 16), lambda i: (i, 0)),
            pl.BlockSpec((_EB, cout), lambda i: (i, 0)),
            _full((cout, 16)),
            _full((1, 16)),
            _full((1, 1)),
        ],
        out_specs=[
            pl.BlockSpec((_EB, cout), lambda i: (i, 0)),
            pl.BlockSpec((_EB, 8), lambda i: (i, 0)),
        ],
        out_shape=[
            jax.ShapeDtypeStruct((_E, cout), f32),
            jax.ShapeDtypeStruct((_E, 8), f32),
        ],
    )(ge, se, eac, w6, b6, b7)


def _sinv(sp):
    def body(s_ref, o_ref):
        s = s_ref[...]
        o_ref[...] = 1.0 / (s[0] + s[1])

    return pl.pallas_call(
        body,
        grid=(_NP // _RPT,),
        in_specs=[pl.BlockSpec((2, _RPT, 8), lambda i: (0, i, 0))],
        out_specs=pl.BlockSpec((_RPT, 8), lambda i: (i, 0)),
        out_shape=jax.ShapeDtypeStruct((_NP, 8), f32),
    )(sp)


def _scale(mp, sg):
    cout = mp.shape[1]
    wpad = max(cout, 8)

    def body(mp_ref, sg_ref, o_ref):
        r = mp_ref[...] * sg_ref[...][:, 0:1]
        if cout < wpad:
            r = jnp.concatenate([r, jnp.zeros((r.shape[0], wpad - cout), f32)], axis=1)
        o_ref[...] = r

    return pl.pallas_call(
        body,
        grid=(_E // _EB,),
        in_specs=[
            pl.BlockSpec((_EB, cout), lambda i: (i, 0)),
            pl.BlockSpec((_EB, 8), lambda i: (i, 0)),
        ],
        out_specs=pl.BlockSpec((_EB, wpad), lambda i: (i, 0)),
        out_shape=jax.ShapeDtypeStruct((_E, wpad), f32),
    )(mp, sg)


def _combine(h, aggp, w2, b2, w3, b3, w4, b4):
    cin = h.shape[1]
    cout = w2.shape[1]

    wpad = max(cout, 8)

    def body(h_ref, a_ref, w2r, b2r, w3r, b3r, w4r, b4r, o_ref):
        a = a_ref[...]
        agg = (a[0] + a[1])[:, :cout]
        xo = jnp.maximum(h_ref[...] @ w2r[...] + b2r[...], 0.0)
        cat = jnp.concatenate([xo, agg], axis=1)
        g1 = jax.nn.sigmoid(cat @ w3r[...] + b3r[...])
        g2 = jax.nn.sigmoid(cat @ w4r[...] + b4r[...])
        o_ref[...] = jnp.maximum(g1 * agg + g2 * xo, 0.0)

    return pl.pallas_call(
        body,
        grid=(_N // _NB,),
        in_specs=[
            pl.BlockSpec((_NB, cin), lambda i: (i, 0)),
            pl.BlockSpec((2, _NB, wpad), lambda i: (0, i, 0)),
            _full((cin, cout)),
            _full((1, cout)),
            _full((2 * cout, 1)),
            _full((1, 1)),
            _full((2 * cout, 1)),
            _full((1, 1)),
        ],
        out_specs=pl.BlockSpec((_NB, cout), lambda i: (i, 0)),
        out_shape=jax.ShapeDtypeStruct((_N, cout), f32),
    )(h, aggp, w2, b2, w3, b3, w4, b4)


def _pool_max(h, batch2, wg, bg):
    c = h.shape[1]

    def body(h_ref, b_ref, wg_ref, bg_ref, m_ref):
        i = pl.program_id(0)
        logit = h_ref[...] @ wg_ref[...] + bg_ref[...]
        oh = b_ref[...] == lax.broadcasted_iota(i32, (_NB, _G), 1)
        masked = jnp.where(oh, logit, -1e38)
        bm = jnp.max(masked, axis=0, keepdims=True)

        @pl.when(i == 0)
        def _():
            m_ref[...] = bm

        @pl.when(i > 0)
        def _():
            m_ref[...] = jnp.maximum(m_ref[...], bm)

    return pl.pallas_call(
        body,
        grid=(_N // _NB,),
        in_specs=[
            pl.BlockSpec((_NB, c), lambda i: (i, 0)),
            pl.BlockSpec((_NB, 1), lambda i: (i, 0)),
            _full((c, 1)),
            _full((1, 1)),
        ],
        out_specs=pl.BlockSpec((1, _G), lambda i: (0, 0)),
        out_shape=jax.ShapeDtypeStruct((1, _G), f32),
    )(h, batch2, wg, bg)


def _pool_sum(h, batch2, wg, bg, m):
    c = h.shape[1]

    def body(h_ref, b_ref, wg_ref, bg_ref, m_ref, num_ref, s_ref):
        i = pl.program_id(0)
        logit = h_ref[...] @ wg_ref[...] + bg_ref[...]
        oh = b_ref[...] == lax.broadcasted_iota(i32, (_NB, _G), 1)
        ohf = oh.astype(f32)
        mnode = lax.dot_general(ohf, m_ref[...], (((1,), (1,)), ((), ())))
        e = jnp.exp(logit - mnode)
        sblk = jnp.sum(ohf * e, axis=0, keepdims=True)
        numblk = lax.dot_general(ohf, e * h_ref[...], (((0,), (0,)), ((), ())))

        @pl.when(i == 0)
        def _():
            num_ref[...] = numblk
            s_ref[...] = sblk

        @pl.when(i > 0)
        def _():
            num_ref[...] = num_ref[...] + numblk
            s_ref[...] = s_ref[...] + sblk

    return pl.pallas_call(
        body,
        grid=(_N // _NB,),
        in_specs=[
            pl.BlockSpec((_NB, c), lambda i: (i, 0)),
            pl.BlockSpec((_NB, 1), lambda i: (i, 0)),
            _full((c, 1)),
            _full((1, 1)),
            _full((1, _G)),
        ],
        out_specs=[
            pl.BlockSpec((_G, c), lambda i: (0, 0)),
            pl.BlockSpec((1, _G), lambda i: (0, 0)),
        ],
        out_shape=[
            jax.ShapeDtypeStruct((_G, c), f32),
            jax.ShapeDtypeStruct((1, _G), f32),
        ],
    )(h, batch2, wg, bg, m)


def _head(num1, s1, num2, s2, params):
    def body(n1, s1r, n2, s2r, w1, b1, w2, b2, w3, b3, w4, b4, o_ref):
        x1 = n1[...] / (jnp.transpose(s1r[...]) + 1e-16)
        x2 = n2[...] / (jnp.transpose(s2r[...]) + 1e-16)
        z = jnp.concatenate([x1, x2], axis=1)
        z = jnp.maximum(z @ w1[...] + b1[...], 0.0)
        z = jnp.maximum(z @ w2[...] + b2[...], 0.0)
        z = jnp.maximum(z @ w3[...] + b3[...], 0.0)
        o_ref[...] = z @ w4[...] + b4[...]

    args = [num1, s1, num2, s2]
    for k in ("lin1", "lin2", "lin3", "lin4"):
        args += [params[k]["w"], params[k]["b"][None, :]]
    return pl.pallas_call(
        body,
        out_shape=jax.ShapeDtypeStruct((_G, 1), f32),
    )(*args)


def _layer(p, h, src, dst, edge_attr, zeros8):
    cin = h.shape[1]
    cout = p["mlp2"]["w"].shape[1]
    w1 = p["mlp1"]["w"]
    wa = w1[:cin]
    wb = w1[cin : 2 * cin]
    wc = w1[2 * cin :]
    kmat = p["em2"]["w"] @ wc
    kbias = (p["em2"]["b"] @ wc + p["mlp1"]["b"])[None, :]
    dt, st = _prep(
        h, wa - wb, wb, p["mlp5"]["w"], p["mlp5"]["b"][None, :], p["mlp7"]["w"][:, 0][None, :]
    )
    eac = _eac(edge_attr, p["em1"]["w"], p["em1"]["b"][None, :], kmat, kbias)
    ge, se = _sc_gather2(dst, src, dt, st)
    mp, pe = _edge_msgp(ge, se, eac, p["mlp6"]["w"], p["mlp6"]["b"][None, :], p["mlp7"]["b"].reshape(1, 1))
    sp = _sc_scatter_add(src, pe, zeros8)
    sinv = _sinv(sp)
    sg = _sc_gather1(src, sinv)
    mps = _scale(mp, sg)
    aggp = _sc_scatter_add(dst, mps, jnp.zeros((_NP, max(cout, 8)), f32))
    return _combine(
        h,
        aggp,
        p["mlp2"]["w"],
        p["mlp2"]["b"][None, :],
        p["mlp3"]["w"],
        p["mlp3"]["b"][None, :],
        p["mlp4"]["w"],
        p["mlp4"]["b"][None, :],
    )


def kernel(x, edge_index, edge_attr, batch, params):
    src = edge_index[0].reshape(_E // _CHUNK, _CHUNK)
    dst = edge_index[1].reshape(_E // _CHUNK, _CHUNK)
    batch2 = batch[:, None]
    zeros8 = jnp.zeros((_NP, 8), f32)

    h = _layer(params["conv1"], x, src, dst, edge_attr, zeros8)
    h = _layer(params["conv2"], h, src, dst, edge_attr, zeros8)
    g1w = params["gate1"]["w"]
    g1b = params["gate1"]["b"][None, :]
    m1 = _pool_max(h, batch2, g1w, g1b)
    n1, s1 = _pool_sum(h, batch2, g1w, g1b, m1)
    h = _layer(params["conv3"], h, src, dst, edge_attr, zeros8)
    h = _layer(params["conv4"], h, src, dst, edge_attr, zeros8)
    g2w = params["gate2"]["w"]
    g2b = params["gate2"]["b"][None, :]
    m2 = _pool_max(h, batch2, g2w, g2b)
    n2, s2 = _pool_sum(h, batch2, g2w, g2b, m2)
    out = _head(n1, s1, n2, s2, params)
    return out[:, 0]


# speedup vs baseline: 6.3957x; 1.0414x over previous
import functools

import jax
import jax.numpy as jnp
from jax import lax
from jax.experimental import pallas as pl
from jax.experimental.pallas import tpu as pltpu
from jax.experimental.pallas import tpu_sc as plsc

_N = 50000
_NP = 50048
_E = 1600000
_G = 64
_NW = 32
_EPW = _E // _NW
_CHUNK = 80
_NCH = _EPW // _CHUNK
_K = 5
_KE = _K * _CHUNK
_NOUT = _EPW // _KE
_RPT = _NP // 16
_NB = 2000
_EB = 4000

f32 = jnp.float32
i32 = jnp.int32


def _mesh():
    return plsc.VectorSubcoreMesh(core_axis_name="c", subcore_axis_name="s")


_SC_PARAMS = pltpu.CompilerParams(use_tc_tiling_on_sc=False)


def _sc_gather2(dst2, src2, dtab, stab):
    dw = dtab.shape[1]
    sw = stab.shape[1]

    def body(dst_h, src_h, dt_h, st_h, ge_h, se_h, dix, six, dbuf, sbuf, si, sg1, sg2, swb):
        wid = lax.axis_index("s") * 2 + lax.axis_index("c")
        base = wid * _EPW
        crow = wid * _NCH

        def idx_load(g, b):
            r0 = crow + g * _K
            return [
                pltpu.async_copy(dst_h.at[pl.ds(r0, _K), :], dix.at[b], si),
                pltpu.async_copy(src_h.at[pl.ds(r0, _K), :], six.at[b], si),
            ]

        def fire(b):
            hs = []
            for j in range(_K):
                hs.append(
                    pltpu.async_copy(
                        dt_h.at[dix.at[b, j]], dbuf.at[b, pl.ds(j * _CHUNK, _CHUNK), :], sg1
                    )
                )
                hs.append(
                    pltpu.async_copy(
                        st_h.at[six.at[b, j]], sbuf.at[b, pl.ds(j * _CHUNK, _CHUNK), :], sg2
                    )
                )
            return hs

        def wb(g, b):
            off = base + g * _KE
            return [
                pltpu.async_copy(dbuf.at[b], ge_h.at[pl.ds(off, _KE), :], swb),
                pltpu.async_copy(sbuf.at[b], se_h.at[pl.ds(off, _KE), :], swb),
            ]

        def it(ip, _):
            g0 = ip * 2
            g1 = g0 + 1
            l0 = idx_load(g0, 0)
            l1 = idx_load(g1, 1)
            for h in l0:
                h.wait()
            f0 = fire(0)
            for h in l1:
                h.wait()
            f1 = fire(1)
            for h in f0:
                h.wait()
            w0 = wb(g0, 0)
            for h in f1:
                h.wait()
            w1 = wb(g1, 1)
            for h in w0 + w1:
                h.wait()
            return 0

        lax.fori_loop(0, _NOUT // 2, it, 0)
        g = _NOUT - 1
        for h in idx_load(g, 0):
            h.wait()
        f0 = fire(0)
        for h in f0:
            h.wait()
        for h in wb(g, 0):
            h.wait()

    return pl.kernel(
        body,
        out_type=[
            jax.ShapeDtypeStruct((_E, dw), f32),
            jax.ShapeDtypeStruct((_E, sw), f32),
        ],
        mesh=_mesh(),
        compiler_params=_SC_PARAMS,
        scratch_types=[
            pltpu.VMEM((2, _K, _CHUNK), i32),
            pltpu.VMEM((2, _K, _CHUNK), i32),
            pltpu.VMEM((2, _KE, dw), f32),
            pltpu.VMEM((2, _KE, sw), f32),
            pltpu.SemaphoreType.DMA,
            pltpu.SemaphoreType.DMA,
            pltpu.SemaphoreType.DMA,
            pltpu.SemaphoreType.DMA,
        ],
    )(dst2, src2, dtab, stab)


def _sc_gather1(idx2, tab):
    w = tab.shape[1]

    def body(idx_h, t_h, o_h, ib, buf, si, sg, swb):
        wid = lax.axis_index("s") * 2 + lax.axis_index("c")
        base = wid * _EPW
        crow = wid * _NCH

        def idx_load(g, b):
            r0 = crow + g * _K
            return [pltpu.async_copy(idx_h.at[pl.ds(r0, _K), :], ib.at[b], si)]

        def fire(b):
            return [
                pltpu.async_copy(
                    t_h.at[ib.at[b, j]], buf.at[b, pl.ds(j * _CHUNK, _CHUNK), :], sg
                )
                for j in range(_K)
            ]

        def wb(g, b):
            off = base + g * _KE
            return [pltpu.async_copy(buf.at[b], o_h.at[pl.ds(off, _KE), :], swb)]

        def it(ip, _):
            g0 = ip * 2
            g1 = g0 + 1
            l0 = idx_load(g0, 0)
            l1 = idx_load(g1, 1)
            for h in l0:
                h.wait()
            f0 = fire(0)
            for h in l1:
                h.wait()
            f1 = fire(1)
            for h in f0:
                h.wait()
            w0 = wb(g0, 0)
            for h in f1:
                h.wait()
            w1 = wb(g1, 1)
            for h in w0 + w1:
                h.wait()
            return 0

        lax.fori_loop(0, _NOUT // 2, it, 0)
        g = _NOUT - 1
        for h in idx_load(g, 0):
            h.wait()
        for h in fire(0):
            h.wait()
        for h in wb(g, 0):
            h.wait()

    return pl.kernel(
        body,
        out_type=jax.ShapeDtypeStruct((_E, w), f32),
        mesh=_mesh(),
        compiler_params=_SC_PARAMS,
        scratch_types=[
            pltpu.VMEM((2, _K, _CHUNK), i32),
            pltpu.VMEM((2, _KE, w), f32),
            pltpu.SemaphoreType.DMA,
            pltpu.SemaphoreType.DMA,
            pltpu.SemaphoreType.DMA,
        ],
    )(idx2, tab)


def _sc_scatter_add(idx2, vals, zeros):
    w = vals.shape[1]

    def body(idx_h, val_h, z_h, out_h, ib, vb, acc_sh, si, sa):
        c = lax.axis_index("c")
        s = lax.axis_index("s")
        wid = s * 2 + c
        r0 = s * _RPT
        pltpu.sync_copy(z_h.at[pl.ds(r0, _RPT), :], acc_sh.at[pl.ds(r0, _RPT), :])
        plsc.subcore_barrier()
        base = wid * _EPW
        crow = wid * _NCH

        def loads(g, b):
            off = base + g * _KE
            rr = crow + g * _K
            return [
                pltpu.async_copy(idx_h.at[pl.ds(rr, _K), :], ib.at[b], si),
                pltpu.async_copy(val_h.at[pl.ds(off, _KE), :], vb.at[b], si),
            ]

        def fire(b):
            return [
                pltpu.async_copy(
                    vb.at[b, pl.ds(j * _CHUNK, _CHUNK), :],
                    acc_sh.at[ib.at[b, j]],
                    sa,
                    add=True,
                )
                for j in range(_K)
            ]

        def it(ip, _):
            g0 = ip * 2
            g1 = g0 + 1
            l0 = loads(g0, 0)
            l1 = loads(g1, 1)
            for h in l0:
                h.wait()
            f0 = fire(0)
            for h in l1:
                h.wait()
            f1 = fire(1)
            for h in f0 + f1:
                h.wait()
            return 0

        lax.fori_loop(0, _NOUT // 2, it, 0)
        g = _NOUT - 1
        for h in loads(g, 0):
            h.wait()
        for h in fire(0):
            h.wait()
        plsc.subcore_barrier()
        pltpu.sync_copy(acc_sh.at[pl.ds(r0, _RPT), :], out_h.at[c, pl.ds(r0, _RPT), :])

    return pl.kernel(
        body,
        out_type=jax.ShapeDtypeStruct((2, _NP, w), f32),
        mesh=_mesh(),
        compiler_params=_SC_PARAMS,
        scratch_types=[
            pltpu.VMEM((2, _K, _CHUNK), i32),
            pltpu.VMEM((2, _KE, w), f32),
            pltpu.VMEM_SHARED((_NP, w), f32),
            pltpu.SemaphoreType.DMA,
            pltpu.SemaphoreType.DMA,
        ],
    )(idx2, vals, zeros)


def _full(shape):
    return pl.BlockSpec(shape, lambda i: tuple(0 for _ in shape))


def _prep(h, wd, wb, w5, b5, w7):
    cin = h.shape[1]
    cout = wd.shape[1]

    def body(h_ref, wd_ref, wb_ref, w5_ref, b5_ref, w7_ref, dt_ref, st_ref):
        hb = h_ref[...]
        an = hb @ wd_ref[...]
        bn = hb @ wb_ref[...]
        w1w7 = jnp.tanh(hb @ w5_ref[...] + b5_ref[...]) * w7_ref[...]
        if cout < 16:
            padd = jnp.zeros((hb.shape[0], 16 - cout), f32)
            dt_ref[...] = jnp.concatenate([an, padd, w1w7], axis=1)
            st_ref[...] = jnp.concatenate([bn, padd], axis=1)
        else:
            dt_ref[...] = jnp.concatenate([an, w1w7], axis=1)
            st_ref[...] = bn

    return pl.pallas_call(
        body,
        grid=(_N // _NB,),
        in_specs=[
            pl.BlockSpec((_NB, cin), lambda i: (i, 0)),
            _full((cin, cout)),
            _full((cin, cout)),
            _full((cin, 16)),
            _full((1, 16)),
            _full((1, 16)),
        ],
        out_specs=[
            pl.BlockSpec((_NB, 32), lambda i: (i, 0)),
            pl.BlockSpec((_NB, 16), lambda i: (i, 0)),
        ],
        out_shape=[
            jax.ShapeDtypeStruct((_N, 32), f32),
            jax.ShapeDtypeStruct((_N, 16), f32),
        ],
    )(h, wd, wb, w5, b5, w7)


def _eac(edge_attr, e1w, e1b, kmat, kbias):
    cout = kmat.shape[1]

    def body(ea_ref, w_ref, b_ref, k_ref, kb_ref, o_ref):
        hrelu = jnp.maximum(ea_ref[...] @ w_ref[...] + b_ref[...], 0.0)
        o_ref[...] = hrelu @ k_ref[...] + kb_ref[...]

    return pl.pallas_call(
        body,
        grid=(_E // _EB,),
        in_specs=[
            pl.BlockSpec((_EB, 3), lambda i: (i, 0)),
            _full((3, 16)),
            _full((1, 16)),
            _full((16, cout)),
            _full((1, cout)),
        ],
        out_specs=pl.BlockSpec((_EB, cout), lambda i: (i, 0)),
        out_shape=jax.ShapeDtypeStruct((_E, cout), f32),
    )(edge_attr, e1w, e1b, kmat, kbias)


def _edge_msgp(ge, se, eac, w6, b6, b7):
    cout = eac.shape[1]

    def body(ge_ref, se_ref, eac_ref, w6_ref, b6_ref, b7_ref, mp_ref, pe_ref):
        a = ge_ref[...]
        msg = jnp.maximum(a[:, :cout] + se_ref[...][:, :cout] + eac_ref[...], 0.0)
        w2 = jnp.tanh(msg @ w6_ref[...] + b6_ref[...])
        logit = jnp.sum(a[:, 16:32] * w2, axis=1, keepdims=True) + b7_ref[...]
        pexp = jnp.exp(logit)
        mp_ref[...] = msg * pexp
        pe_ref[...] = jnp.broadcast_to(pexp, (pexp.shape[0], 8))

    return pl.pallas_call(
        body,
        grid=(_E // _EB,),
        in_specs=[
            pl.BlockSpec((_EB, 32), lambda i: (i, 0)),
            pl.BlockSpec((_EB, 16), lambda i: (i, 0)),
            pl.BlockSpec((_EB, cout), lambda i: (i, 0)),
            _full((cout, 16)),
            _full((1, 16)),
            _full((1, 1)),
        ],
        out_specs=[
            pl.BlockSpec((_EB, cout), lambda i: (i, 0)),
            pl.BlockSpec((_EB, 8), lambda i: (i, 0)),
        ],
        out_shape=[
            jax.ShapeDtypeStruct((_E, cout), f32),
            jax.ShapeDtypeStruct((_E, 8), f32),
        ],
    )(ge, se, eac, w6, b6, b7)


def _sinv(sp):
    def body(s_ref, o_ref):
        s = s_ref[...]
        o_ref[...] = 1.0 / (s[0] + s[1])

    return pl.pallas_call(
        body,
        grid=(_NP // _RPT,),
        in_specs=[pl.BlockSpec((2, _RPT, 8), lambda i: (0, i, 0))],
        out_specs=pl.BlockSpec((_RPT, 8), lambda i: (i, 0)),
        out_shape=jax.ShapeDtypeStruct((_NP, 8), f32),
    )(sp)


def _scale(mp, sg):
    cout = mp.shape[1]
    wpad = max(cout, 8)

    def body(mp_ref, sg_ref, o_ref):
        r = mp_ref[...] * sg_ref[...][:, 0:1]
        if cout < wpad:
            r = jnp.concatenate([r, jnp.zeros((r.shape[0], wpad - cout), f32)], axis=1)
        o_ref[...] = r

    return pl.pallas_call(
        body,
        grid=(_E // _EB,),
        in_specs=[
            pl.BlockSpec((_EB, cout), lambda i: (i, 0)),
            pl.BlockSpec((_EB, 8), lambda i: (i, 0)),
        ],
        out_specs=pl.BlockSpec((_EB, wpad), lambda i: (i, 0)),
        out_shape=jax.ShapeDtypeStruct((_E, wpad), f32),
    )(mp, sg)


def _combine(h, aggp, w2, b2, w3, b3, w4, b4):
    cin = h.shape[1]
    cout = w2.shape[1]

    wpad = max(cout, 8)

    def body(h_ref, a_ref, w2r, b2r, w3r, b3r, w4r, b4r, o_ref):
        a = a_ref[...]
        agg = (a[0] + a[1])[:, :cout]
        xo = jnp.maximum(h_ref[...] @ w2r[...] + b2r[...], 0.0)
        cat = jnp.concatenate([xo, agg], axis=1)
        g1 = jax.nn.sigmoid(cat @ w3r[...] + b3r[...])
        g2 = jax.nn.sigmoid(cat @ w4r[...] + b4r[...])
        o_ref[...] = jnp.maximum(g1 * agg + g2 * xo, 0.0)

    return pl.pallas_call(
        body,
        grid=(_N // _NB,),
        in_specs=[
            pl.BlockSpec((_NB, cin), lambda i: (i, 0)),
            pl.BlockSpec((2, _NB, wpad), lambda i: (0, i, 0)),
            _full((cin, cout)),
            _full((1, cout)),
            _full((2 * cout, 1)),
            _full((1, 1)),
            _full((2 * cout, 1)),
            _full((1, 1)),
        ],
        out_specs=pl.BlockSpec((_NB, cout), lambda i: (i, 0)),
        out_shape=jax.ShapeDtypeStruct((_N, cout), f32),
    )(h, aggp, w2, b2, w3, b3, w4, b4)


def _pool_max(h, batch2, wg, bg):
    c = h.shape[1]

    def body(h_ref, b_ref, wg_ref, bg_ref, m_ref):
        i = pl.program_id(0)
        logit = h_ref[...] @ wg_ref[...] + bg_ref[...]
        oh = b_ref[...] == lax.broadcasted_iota(i32, (_NB, _G), 1)
        masked = jnp.where(oh, logit, -1e38)
        bm = jnp.max(masked, axis=0, keepdims=True)

        @pl.when(i == 0)
        def _():
            m_ref[...] = bm

        @pl.when(i > 0)
        def _():
            m_ref[...] = jnp.maximum(m_ref[...], bm)

    return pl.pallas_call(
        body,
        grid=(_N // _NB,),
        in_specs=[
            pl.BlockSpec((_NB, c), lambda i: (i, 0)),
            pl.BlockSpec((_NB, 1), lambda i: (i, 0)),
            _full((c, 1)),
            _full((1, 1)),
        ],
        out_specs=pl.BlockSpec((1, _G), lambda i: (0, 0)),
        out_shape=jax.ShapeDtypeStruct((1, _G), f32),
    )(h, batch2, wg, bg)


def _pool_sum(h, batch2, wg, bg, m):
    c = h.shape[1]

    def body(h_ref, b_ref, wg_ref, bg_ref, m_ref, num_ref, s_ref):
        i = pl.program_id(0)
        logit = h_ref[...] @ wg_ref[...] + bg_ref[...]
        oh = b_ref[...] == lax.broadcasted_iota(i32, (_NB, _G), 1)
        ohf = oh.astype(f32)
        mnode = lax.dot_general(ohf, m_ref[...], (((1,), (1,)), ((), ())))
        e = jnp.exp(logit - mnode)
        sblk = jnp.sum(ohf * e, axis=0, keepdims=True)
        numblk = lax.dot_general(ohf, e * h_ref[...], (((0,), (0,)), ((), ())))

        @pl.when(i == 0)
        def _():
            num_ref[...] = numblk
            s_ref[...] = sblk

        @pl.when(i > 0)
        def _():
            num_ref[...] = num_ref[...] + numblk
            s_ref[...] = s_ref[...] + sblk

    return pl.pallas_call(
        body,
        grid=(_N // _NB,),
        in_specs=[
            pl.BlockSpec((_NB, c), lambda i: (i, 0)),
            pl.BlockSpec((_NB, 1), lambda i: (i, 0)),
            _full((c, 1)),
            _full((1, 1)),
            _full((1, _G)),
        ],
        out_specs=[
            pl.BlockSpec((_G, c), lambda i: (0, 0)),
            pl.BlockSpec((1, _G), lambda i: (0, 0)),
        ],
        out_shape=[
            jax.ShapeDtypeStruct((_G, c), f32),
            jax.ShapeDtypeStruct((1, _G), f32),
        ],
    )(h, batch2, wg, bg, m)


def _head(num1, s1, num2, s2, params):
    def body(n1, s1r, n2, s2r, w1, b1, w2, b2, w3, b3, w4, b4, o_ref):
        x1 = n1[...] / (jnp.transpose(s1r[...]) + 1e-16)
        x2 = n2[...] / (jnp.transpose(s2r[...]) + 1e-16)
        z = jnp.concatenate([x1, x2], axis=1)
        z = jnp.maximum(z @ w1[...] + b1[...], 0.0)
        z = jnp.maximum(z @ w2[...] + b2[...], 0.0)
        z = jnp.maximum(z @ w3[...] + b3[...], 0.0)
        o_ref[...] = z @ w4[...] + b4[...]

    args = [num1, s1, num2, s2]
    for k in ("lin1", "lin2", "lin3", "lin4"):
        args += [params[k]["w"], params[k]["b"][None, :]]
    return pl.pallas_call(
        body,
        out_shape=jax.ShapeDtypeStruct((_G, 1), f32),
    )(*args)


def _layer(p, h, src, dst, edge_attr, zeros8):
    cin = h.shape[1]
    cout = p["mlp2"]["w"].shape[1]
    w1 = p["mlp1"]["w"]
    wa = w1[:cin]
    wb = w1[cin : 2 * cin]
    wc = w1[2 * cin :]
    kmat = p["em2"]["w"] @ wc
    kbias = (p["em2"]["b"] @ wc + p["mlp1"]["b"])[None, :]
    dt, st = _prep(
        h, wa - wb, wb, p["mlp5"]["w"], p["mlp5"]["b"][None, :], p["mlp7"]["w"][:, 0][None, :]
    )
    eac = _eac(edge_attr, p["em1"]["w"], p["em1"]["b"][None, :], kmat, kbias)
    ge, se = _sc_gather2(dst, src, dt, st)
    mp, pe = _edge_msgp(ge, se, eac, p["mlp6"]["w"], p["mlp6"]["b"][None, :], p["mlp7"]["b"].reshape(1, 1))
    sp = _sc_scatter_add(src, pe, zeros8)
    sinv = _sinv(sp)
    sg = _sc_gather1(src, sinv)
    mps = _scale(mp, sg)
    aggp = _sc_scatter_add(dst, mps, jnp.zeros((_NP, max(cout, 8)), f32))
    return _combine(
        h,
        aggp,
        p["mlp2"]["w"],
        p["mlp2"]["b"][None, :],
        p["mlp3"]["w"],
        p["mlp3"]["b"][None, :],
        p["mlp4"]["w"],
        p["mlp4"]["b"][None, :],
    )


def kernel(x, edge_index, edge_attr, batch, params):
    src = edge_index[0].reshape(_E // _CHUNK, _CHUNK)
    dst = edge_index[1].reshape(_E // _CHUNK, _CHUNK)
    batch2 = batch[:, None]
    zeros8 = jnp.zeros((_NP, 8), f32)

    h = _layer(params["conv1"], x, src, dst, edge_attr, zeros8)
    h = _layer(params["conv2"], h, src, dst, edge_attr, zeros8)
    g1w = params["gate1"]["w"]
    g1b = params["gate1"]["b"][None, :]
    m1 = _pool_max(h, batch2, g1w, g1b)
    n1, s1 = _pool_sum(h, batch2, g1w, g1b, m1)
    h = _layer(params["conv3"], h, src, dst, edge_attr, zeros8)
    h = _layer(params["conv4"], h, src, dst, edge_attr, zeros8)
    g2w = params["gate2"]["w"]
    g2b = params["gate2"]["b"][None, :]
    m2 = _pool_max(h, batch2, g2w, g2b)
    n2, s2 = _pool_sum(h, batch2, g2w, g2b, m2)
    out = _head(n1, s1, n2, s2, params)
    return out[:, 0]
